# Initial kernel scaffold; baseline (speedup 1.0000x reference)
import jax
import jax.numpy as jnp
from jax.experimental import pallas as pl


def kernel(node_embeddings, adjacency_lists, Wr1, Ws1, b1, Wr2, Ws2, b2):
    raise NotImplementedError("write your pallas kernel here")


# speedup vs baseline: 1.4932x; 1.4932x over previous
import jax
import jax.numpy as jnp
from jax import lax
from jax.experimental import pallas as pl
from jax.experimental.pallas import tpu as pltpu
from jax.experimental.pallas import tpu_sc as plsc

N_NODES = 10000
D_FEAT = 256
HIDDEN = 256
NUM_CLASS = 16
N_REL = 4
E_PER_REL = 40000

NC, NS, LANES = 2, 16, 16
QW = 64
NQ = 4
QPC = NQ // NC
N_PAD = 10240
E_PAD = 40960
EDGES_PER_TILE = E_PAD // NS
CHUNK = 128
CHUNKS_PER_TILE = EDGES_PER_TILE // CHUNK
ROWS_PER_TILE = N_PAD // NS
TRASH_ROW = N_NODES
DR = N_PAD // 128


def _sc_agg_body(x_hbm, src_hbm, dst_hbm, zf_hbm, zc_hbm,
                 s_out, deg_out,
                 src_all, dst_all, gidx, sidx, rows, zbuf, cnt, iidx,
                 accum, dacc, sem):
    c = lax.axis_index("c")
    s = lax.axis_index("s")
    row0 = s * ROWS_PER_TILE
    drow0 = s * (DR // NS)

    pltpu.sync_copy(zf_hbm, zbuf)
    for k in range(DR // LANES):
        iidx[pl.ds(k * LANES, LANES)] = (
            jax.lax.iota(jnp.int32, LANES) + k * LANES)

    for r in range(N_REL):
        pltpu.sync_copy(src_hbm.at[r, pl.ds(s * EDGES_PER_TILE, EDGES_PER_TILE)],
                        src_all)
        pltpu.sync_copy(dst_hbm.at[r, pl.ds(s * EDGES_PER_TILE, EDGES_PER_TILE)],
                        dst_all)

        for p in range(QPC):
            q_off = (c * QPC + p) * N_PAD
            count_pass = p == 0

            pltpu.sync_copy(zbuf, accum.at[pl.ds(row0, ROWS_PER_TILE)])
            if count_pass:
                @pl.when(c == 0)
                def _():
                    pltpu.sync_copy(zc_hbm, cnt)

                @pl.when(jnp.logical_and(c == 0, s == 0))
                def _():
                    pltpu.sync_copy(zc_hbm, dacc)
            plsc.subcore_barrier()

            def chunk_body(j, carry):
                base = j * CHUNK
                for k in range(CHUNK // LANES):
                    off = base + k * LANES
                    dv = dst_all[pl.ds(off, LANES)]
                    gidx[pl.ds(k * LANES, LANES)] = (
                        src_all[pl.ds(off, LANES)] + q_off)
                    sidx[pl.ds(k * LANES, LANES)] = dv
                    if count_pass:
                        @pl.when(c == 0)
                        def _():
                            plsc.addupdate_scatter(
                                cnt,
                                [lax.shift_right_logical(dv, 7),
                                 lax.bitwise_and(dv, 127)],
                                jnp.ones((LANES,), jnp.float32))
                pltpu.async_copy(x_hbm.at[gidx], rows, sem).wait()
                pltpu.sync_copy(rows, accum.at[sidx], add=True)
                return carry

            lax.fori_loop(0, CHUNKS_PER_TILE, chunk_body, 0)
            plsc.subcore_barrier()

            pltpu.sync_copy(
                accum.at[pl.ds(row0, ROWS_PER_TILE)],
                s_out.at[r, pl.ds((c * QPC + p) * N_PAD + row0, ROWS_PER_TILE)])

            if count_pass:
                @pl.when(c == 0)
                def _():
                    pltpu.sync_copy(cnt, dacc.at[iidx], add=True)
                plsc.subcore_barrier()

                @pl.when(jnp.logical_and(c == 0, s == 0))
                def _():
                    pltpu.sync_copy(dacc, deg_out.at[r])


def _sc_agg(x_quart, src_pad, dst_pad, zf, zc):
    mesh = plsc.VectorSubcoreMesh(core_axis_name="c", subcore_axis_name="s",
                                  num_cores=NC, num_subcores=NS)
    return pl.kernel(
        _sc_agg_body,
        out_type=[
            jax.ShapeDtypeStruct((N_REL, NQ * N_PAD, QW), jnp.float32),
            jax.ShapeDtypeStruct((N_REL, DR, 128), jnp.float32),
        ],
        mesh=mesh,
        compiler_params=pltpu.CompilerParams(needs_layout_passes=False,
                                             use_tc_tiling_on_sc=False),
        scratch_types=[
            pltpu.VMEM((EDGES_PER_TILE,), jnp.int32),
            pltpu.VMEM((EDGES_PER_TILE,), jnp.int32),
            pltpu.VMEM((CHUNK,), jnp.int32),
            pltpu.VMEM((CHUNK,), jnp.int32),
            pltpu.VMEM((CHUNK, QW), jnp.float32),
            pltpu.VMEM((ROWS_PER_TILE, QW), jnp.float32),
            pltpu.VMEM((DR, 128), jnp.float32),
            pltpu.VMEM((DR,), jnp.int32),
            pltpu.VMEM_SHARED((N_PAD, QW), jnp.float32),
            pltpu.VMEM_SHARED((DR, 128), jnp.float32),
            pltpu.SemaphoreType.DMA,
        ],
    )(x_quart, src_pad, dst_pad, zf, zc)


def _tc_layer1_body(x_ref, s_ref, deg_ref, ws_ref, wr_ref, b_ref, out_ref):
    acc = jnp.dot(x_ref[...], ws_ref[...], preferred_element_type=jnp.float32)
    acc += b_ref[...]
    for r in range(N_REL):
        s_full = jnp.concatenate([s_ref[r, q] for q in range(NQ)], axis=-1)
        m = jnp.dot(s_full, wr_ref[r], preferred_element_type=jnp.float32)
        acc += m / jnp.maximum(deg_ref[r], 1.0)
    h = jnp.maximum(acc, 0.0)
    for q in range(NQ):
        out_ref[q] = h[:, q * QW:(q + 1) * QW]


def _tc_layer2_body(h_ref, s_ref, deg_ref, ws_ref, wr_ref, b_ref, out_ref):
    hx = jnp.concatenate([h_ref[q] for q in range(NQ)], axis=-1)
    acc = jnp.dot(hx, ws_ref[...], preferred_element_type=jnp.float32)
    acc += b_ref[...]
    for r in range(N_REL):
        s_full = jnp.concatenate([s_ref[r, q] for q in range(NQ)], axis=-1)
        m = jnp.dot(s_full, wr_ref[r], preferred_element_type=jnp.float32)
        acc += m / jnp.maximum(deg_ref[r], 1.0)
    mx = jnp.max(acc, axis=1, keepdims=True)
    e = jnp.exp(acc - mx)
    out_ref[...] = e / jnp.sum(e, axis=1, keepdims=True)


_BLK = 1024


def _tc_layer1(x, s1, deg, ws1, wr1, b1):
    return pl.pallas_call(
        _tc_layer1_body,
        grid=(N_PAD // _BLK,),
        in_specs=[
            pl.BlockSpec((_BLK, D_FEAT), lambda i: (i, 0)),
            pl.BlockSpec((N_REL, NQ, _BLK, QW), lambda i: (0, 0, i, 0)),
            pl.BlockSpec((N_REL, _BLK, 1), lambda i: (0, i, 0)),
            pl.BlockSpec((D_FEAT, HIDDEN), lambda i: (0, 0)),
            pl.BlockSpec((N_REL, D_FEAT, HIDDEN), lambda i: (0, 0, 0)),
            pl.BlockSpec((1, HIDDEN), lambda i: (0, 0)),
        ],
        out_specs=pl.BlockSpec((NQ, _BLK, QW), lambda i: (0, i, 0)),
        out_shape=jax.ShapeDtypeStruct((NQ, N_PAD, QW), jnp.float32),
    )(x, s1, deg, ws1, wr1, b1)


def _tc_layer2(h_quart, s2, deg, ws2, wr2, b2):
    return pl.pallas_call(
        _tc_layer2_body,
        grid=(N_PAD // _BLK,),
        in_specs=[
            pl.BlockSpec((NQ, _BLK, QW), lambda i: (0, i, 0)),
            pl.BlockSpec((N_REL, NQ, _BLK, QW), lambda i: (0, 0, i, 0)),
            pl.BlockSpec((N_REL, _BLK, 1), lambda i: (0, i, 0)),
            pl.BlockSpec((HIDDEN, NUM_CLASS), lambda i: (0, 0)),
            pl.BlockSpec((N_REL, HIDDEN, NUM_CLASS), lambda i: (0, 0, 0)),
            pl.BlockSpec((1, NUM_CLASS), lambda i: (0, 0)),
        ],
        out_specs=pl.BlockSpec((_BLK, NUM_CLASS), lambda i: (i, 0)),
        out_shape=jax.ShapeDtypeStruct((N_PAD, NUM_CLASS), jnp.float32),
    )(h_quart, s2, deg, ws2, wr2, b2)


def kernel(node_embeddings, adjacency_lists, Wr1, Ws1, b1, Wr2, Ws2, b2):
    adj = adjacency_lists.astype(jnp.int32)
    src = jnp.pad(adj[:, :, 0], ((0, 0), (0, E_PAD - E_PER_REL)))
    dst = jnp.pad(adj[:, :, 1], ((0, 0), (0, E_PAD - E_PER_REL)),
                  constant_values=TRASH_ROW)
    zf = jnp.zeros((ROWS_PER_TILE, QW), jnp.float32)
    zc = jnp.zeros((DR, 128), jnp.float32)

    x_pad = jnp.pad(node_embeddings, ((0, N_PAD - N_NODES), (0, 0)))
    x_quart = (x_pad.reshape(N_PAD, NQ, QW).transpose(1, 0, 2)
               .reshape(NQ * N_PAD, QW))

    s1_flat, deg_r = _sc_agg(x_quart, src, dst, zf, zc)
    s1 = s1_flat.reshape(N_REL, NQ, N_PAD, QW)
    deg = deg_r.reshape(N_REL, N_PAD, 1)

    h_quart = _tc_layer1(x_pad, s1, deg, Ws1, Wr1, b1.reshape(1, HIDDEN))

    s2_flat, _ = _sc_agg(h_quart.reshape(NQ * N_PAD, QW), src, dst, zf, zc)
    s2 = s2_flat.reshape(N_REL, NQ, N_PAD, QW)

    out = _tc_layer2(h_quart, s2, deg, Ws2, Wr2, b2.reshape(1, NUM_CLASS))
    return out[:N_NODES]


# speedup vs baseline: 1.5597x; 1.0445x over previous
import jax
import jax.numpy as jnp
from jax import lax
from jax.experimental import pallas as pl
from jax.experimental.pallas import tpu as pltpu
from jax.experimental.pallas import tpu_sc as plsc

N_NODES = 10000
D_FEAT = 256
HIDDEN = 256
NUM_CLASS = 16
N_REL = 4
E_PER_REL = 40000

NC, NS, LANES = 2, 16, 16
QW = 64
NQ = 4
QPC = NQ // NC
N_PAD = 10240
E_PAD = 40960
EDGES_PER_TILE = E_PAD // NS
CHUNK = 128
CHUNKS_PER_TILE = EDGES_PER_TILE // CHUNK
ROWS_PER_TILE = N_PAD // NS
TRASH_ROW = N_NODES
DR = N_PAD // 128


NBUF = 2
GROUP = NBUF * CHUNK


def _sc_agg_body(x_hbm, src_hbm, dst_hbm, zf_hbm, zc_hbm,
                 s_out, deg_out,
                 src_grp, dst_grp,
                 gidx0, gidx1, sidx0, sidx1, rows0, rows1,
                 cnt, iidx, accum, dacc,
                 gsem0, gsem1, ssem0, ssem1, isem0, isem1):
    gidxs = (gidx0, gidx1)
    sidxs = (sidx0, sidx1)
    rowss = (rows0, rows1)
    gsems = (gsem0, gsem1)
    ssems = (ssem0, ssem1)
    c = lax.axis_index("c")
    s = lax.axis_index("s")
    row0 = s * ROWS_PER_TILE
    e0 = s * EDGES_PER_TILE

    for k in range(DR // LANES):
        iidx[pl.ds(k * LANES, LANES)] = (
            jax.lax.iota(jnp.int32, LANES) + k * LANES)

    for r in range(N_REL):
        for p in range(QPC):
            q_off = (c * QPC + p) * N_PAD
            count_pass = p == 0

            pltpu.sync_copy(zf_hbm, accum.at[pl.ds(row0, ROWS_PER_TILE)])
            if count_pass:
                @pl.when(c == 0)
                def _():
                    pltpu.sync_copy(zc_hbm, cnt)

                @pl.when(jnp.logical_and(c == 0, s == 0))
                def _():
                    pltpu.sync_copy(zc_hbm, dacc)
            plsc.subcore_barrier()

            def chunk_group(jj, carry):
                base0 = e0 + jj * GROUP
                i0 = pltpu.async_copy(
                    src_hbm.at[r, pl.ds(base0, CHUNK)], src_grp, isem0)
                i1 = pltpu.async_copy(
                    dst_hbm.at[r, pl.ds(base0, GROUP)], dst_grp, isem1)
                i0.wait()
                i1.wait()
                gd = []
                for b in range(NBUF):
                    for k in range(CHUNK // LANES):
                        off = b * CHUNK + k * LANES
                        dv = dst_grp[pl.ds(off, LANES)]
                        if b == 0:
                            gidxs[0][pl.ds(k * LANES, LANES)] = (
                                src_grp[pl.ds(k * LANES, LANES)] + q_off)
                        sidxs[b][pl.ds(k * LANES, LANES)] = dv
                        if count_pass:
                            @pl.when(c == 0)
                            def _():
                                plsc.addupdate_scatter(
                                    cnt,
                                    [lax.shift_right_logical(dv, 7),
                                     lax.bitwise_and(dv, 127)],
                                    jnp.ones((LANES,), jnp.float32))
                    if b == 0:
                        gd.append(pltpu.async_copy(x_hbm.at[gidxs[0]],
                                                   rowss[0], gsems[0]))
                        pltpu.async_copy(
                            src_hbm.at[r, pl.ds(base0 + CHUNK, CHUNK)],
                            src_grp, isem0).wait()
                        for k in range(CHUNK // LANES):
                            gidxs[1][pl.ds(k * LANES, LANES)] = (
                                src_grp[pl.ds(k * LANES, LANES)] + q_off)
                        gd.append(pltpu.async_copy(x_hbm.at[gidxs[1]],
                                                   rowss[1], gsems[1]))
                sd = []
                for b in range(NBUF):
                    gd[b].wait()
                    sd.append(pltpu.async_copy(rowss[b], accum.at[sidxs[b]],
                                               ssems[b], add=True))
                for b in range(NBUF):
                    sd[b].wait()
                return carry

            lax.fori_loop(0, CHUNKS_PER_TILE // NBUF, chunk_group, 0)
            plsc.subcore_barrier()

            pltpu.sync_copy(
                accum.at[pl.ds(row0, ROWS_PER_TILE)],
                s_out.at[r, pl.ds((c * QPC + p) * N_PAD + row0, ROWS_PER_TILE)])

            if count_pass:
                @pl.when(c == 0)
                def _():
                    pltpu.sync_copy(cnt, dacc.at[iidx], add=True)
                plsc.subcore_barrier()

                @pl.when(jnp.logical_and(c == 0, s == 0))
                def _():
                    pltpu.sync_copy(dacc, deg_out.at[r])


def _sc_agg(x_quart, src_pad, dst_pad, zf, zc):
    mesh = plsc.VectorSubcoreMesh(core_axis_name="c", subcore_axis_name="s",
                                  num_cores=NC, num_subcores=NS)
    return pl.kernel(
        _sc_agg_body,
        out_type=[
            jax.ShapeDtypeStruct((N_REL, NQ * N_PAD, QW), jnp.float32),
            jax.ShapeDtypeStruct((N_REL, DR, 128), jnp.float32),
        ],
        mesh=mesh,
        compiler_params=pltpu.CompilerParams(needs_layout_passes=False,
                                             use_tc_tiling_on_sc=False),
        scratch_types=[
            pltpu.VMEM((CHUNK,), jnp.int32),
            pltpu.VMEM((GROUP,), jnp.int32),
            *[pltpu.VMEM((CHUNK,), jnp.int32) for _ in range(2 * NBUF)],
            *[pltpu.VMEM((CHUNK, QW), jnp.float32) for _ in range(NBUF)],
            pltpu.VMEM((DR, 128), jnp.float32),
            pltpu.VMEM((DR,), jnp.int32),
            pltpu.VMEM_SHARED((N_PAD, QW), jnp.float32),
            pltpu.VMEM_SHARED((DR, 128), jnp.float32),
            *[pltpu.SemaphoreType.DMA for _ in range(2 * NBUF + 2)],
        ],
    )(x_quart, src_pad, dst_pad, zf, zc)


def _tc_layer1_body(x_ref, s_ref, deg_ref, ws_ref, wr_ref, b_ref, out_ref):
    acc = jnp.dot(x_ref[...], ws_ref[...], preferred_element_type=jnp.float32)
    acc += b_ref[...]
    for r in range(N_REL):
        s_full = jnp.concatenate([s_ref[r, q] for q in range(NQ)], axis=-1)
        m = jnp.dot(s_full, wr_ref[r], preferred_element_type=jnp.float32)
        acc += m / jnp.maximum(deg_ref[r], 1.0)
    h = jnp.maximum(acc, 0.0)
    for q in range(NQ):
        out_ref[q] = h[:, q * QW:(q + 1) * QW]


def _tc_layer2_body(h_ref, s_ref, deg_ref, ws_ref, wr_ref, b_ref, out_ref):
    hx = jnp.concatenate([h_ref[q] for q in range(NQ)], axis=-1)
    acc = jnp.dot(hx, ws_ref[...], preferred_element_type=jnp.float32)
    acc += b_ref[...]
    for r in range(N_REL):
        s_full = jnp.concatenate([s_ref[r, q] for q in range(NQ)], axis=-1)
        m = jnp.dot(s_full, wr_ref[r], preferred_element_type=jnp.float32)
        acc += m / jnp.maximum(deg_ref[r], 1.0)
    mx = jnp.max(acc, axis=1, keepdims=True)
    e = jnp.exp(acc - mx)
    out_ref[...] = e / jnp.sum(e, axis=1, keepdims=True)


_BLK = 1024


def _tc_layer1(x, s1, deg, ws1, wr1, b1):
    return pl.pallas_call(
        _tc_layer1_body,
        grid=(N_PAD // _BLK,),
        in_specs=[
            pl.BlockSpec((_BLK, D_FEAT), lambda i: (i, 0)),
            pl.BlockSpec((N_REL, NQ, _BLK, QW), lambda i: (0, 0, i, 0)),
            pl.BlockSpec((N_REL, _BLK, 1), lambda i: (0, i, 0)),
            pl.BlockSpec((D_FEAT, HIDDEN), lambda i: (0, 0)),
            pl.BlockSpec((N_REL, D_FEAT, HIDDEN), lambda i: (0, 0, 0)),
            pl.BlockSpec((1, HIDDEN), lambda i: (0, 0)),
        ],
        out_specs=pl.BlockSpec((NQ, _BLK, QW), lambda i: (0, i, 0)),
        out_shape=jax.ShapeDtypeStruct((NQ, N_PAD, QW), jnp.float32),
    )(x, s1, deg, ws1, wr1, b1)


def _tc_layer2(h_quart, s2, deg, ws2, wr2, b2):
    return pl.pallas_call(
        _tc_layer2_body,
        grid=(N_PAD // _BLK,),
        in_specs=[
            pl.BlockSpec((NQ, _BLK, QW), lambda i: (0, i, 0)),
            pl.BlockSpec((N_REL, NQ, _BLK, QW), lambda i: (0, 0, i, 0)),
            pl.BlockSpec((N_REL, _BLK, 1), lambda i: (0, i, 0)),
            pl.BlockSpec((HIDDEN, NUM_CLASS), lambda i: (0, 0)),
            pl.BlockSpec((N_REL, HIDDEN, NUM_CLASS), lambda i: (0, 0, 0)),
            pl.BlockSpec((1, NUM_CLASS), lambda i: (0, 0)),
        ],
        out_specs=pl.BlockSpec((_BLK, NUM_CLASS), lambda i: (i, 0)),
        out_shape=jax.ShapeDtypeStruct((N_PAD, NUM_CLASS), jnp.float32),
    )(h_quart, s2, deg, ws2, wr2, b2)


def kernel(node_embeddings, adjacency_lists, Wr1, Ws1, b1, Wr2, Ws2, b2):
    adj = adjacency_lists.astype(jnp.int32)
    src = jnp.pad(adj[:, :, 0], ((0, 0), (0, E_PAD - E_PER_REL)))
    dst = jnp.pad(adj[:, :, 1], ((0, 0), (0, E_PAD - E_PER_REL)),
                  constant_values=TRASH_ROW)
    zf = jnp.zeros((ROWS_PER_TILE, QW), jnp.float32)
    zc = jnp.zeros((DR, 128), jnp.float32)

    x_pad = jnp.pad(node_embeddings, ((0, N_PAD - N_NODES), (0, 0)))
    x_quart = (x_pad.reshape(N_PAD, NQ, QW).transpose(1, 0, 2)
               .reshape(NQ * N_PAD, QW))

    s1_flat, deg_r = _sc_agg(x_quart, src, dst, zf, zc)
    s1 = s1_flat.reshape(N_REL, NQ, N_PAD, QW)
    deg = deg_r.reshape(N_REL, N_PAD, 1)

    h_quart = _tc_layer1(x_pad, s1, deg, Ws1, Wr1, b1.reshape(1, HIDDEN))

    s2_flat, _ = _sc_agg(h_quart.reshape(NQ * N_PAD, QW), src, dst, zf, zc)
    s2 = s2_flat.reshape(N_REL, NQ, N_PAD, QW)

    out = _tc_layer2(h_quart, s2, deg, Ws2, Wr2, b2.reshape(1, NUM_CLASS))
    return out[:N_NODES]
